# baseline (speedup 1.0000x reference)
import functools

import jax
import jax.numpy as jnp
from jax import lax
from jax.experimental import pallas as pl
from jax.experimental.pallas import tpu as pltpu
from jax.experimental.pallas import tpu_sc as plsc

TOTAL = 16384
NSEG = 16
D = 1024
CSC = 512
SC_ROWS = 4096
RPW = SC_ROWS // 16
CH = 64
NCH = RPW // CH
NG = CSC // 16
NEG = float("-inf")


def _reduce_chunk_ragged(buf, acc, r0, starts_v, ends_v):
    for s in range(NSEG):
        lo = jnp.clip(starts_v[s] - r0, 0, CH)
        hi = jnp.clip(ends_v[s] - r0, 0, CH)

        @pl.when(hi > lo)
        def _process(s=s, lo=lo, hi=hi):
            for h in range(2):
                hg = NG // 2
                c0 = h * hg * 16
                accv = tuple(
                    acc[s, pl.ds(c0 + g * 16, 16)] for g in range(hg)
                )

                def row_body(r, c, c0=c0, hg=hg):
                    return tuple(
                        jnp.maximum(c[g], buf[r, pl.ds(c0 + g * 16, 16)])
                        for g in range(hg)
                    )

                accv = lax.fori_loop(lo, hi, row_body, accv)
                for g in range(hg):
                    acc[s, pl.ds(c0 + g * 16, 16)] = accv[g]


def _reduce_chunk(buf, acc, r0, starts_v, ends_v):
    flags = [
        (starts_v[s] <= r0) & (ends_v[s] >= r0 + CH) for s in range(NSEG)
    ]
    any_full = functools.reduce(jnp.logical_or, flags)
    s_full = functools.reduce(
        jnp.add,
        [jnp.where(flags[s], jnp.int32(s), jnp.int32(0)) for s in range(NSEG)],
    )

    @pl.when(any_full)
    def _fast():
        for h in range(2):
            hg = NG // 2
            c0 = h * hg * 16
            mv = tuple(buf[0, pl.ds(c0 + g * 16, 16)] for g in range(hg))

            @plsc.parallel_loop(1, CH, unroll=4, carry=mv)
            def mv(r, c, c0=c0, hg=hg):
                return tuple(
                    jnp.maximum(c[g], buf[r, pl.ds(c0 + g * 16, 16)])
                    for g in range(hg)
                )

            for g in range(hg):
                a = acc[s_full, pl.ds(c0 + g * 16, 16)]
                acc[s_full, pl.ds(c0 + g * 16, 16)] = jnp.maximum(a, mv[g])

    @pl.when(jnp.logical_not(any_full))
    def _slow():
        _reduce_chunk_ragged(buf, acc, r0, starts_v, ends_v)


def _body(x_hbm, starts_hbm, ends_hbm, out_hbm,
          bnds_s, buf0, buf1, acc, mbuf, macc, spmem, sem0, sem1):
    cid = lax.axis_index("c")
    sid = lax.axis_index("s")
    col0 = cid * CSC
    row0 = sid * RPW

    def chunk_src(i):
        return x_hbm.at[pl.ds(row0 + i * CH, CH), pl.ds(col0, CSC)]

    pltpu.sync_copy(starts_hbm, bnds_s.at[0])
    pltpu.sync_copy(ends_hbm, bnds_s.at[1])
    starts_v = bnds_s[0]
    ends_v = bnds_s[1]

    pltpu.async_copy(chunk_src(0), buf0, sem0)

    neg = jnp.full((16,), NEG, jnp.float32)
    for s in range(NSEG):
        for g in range(NG):
            acc[s, pl.ds(g * 16, 16)] = neg

    def pair_body(g, carry):
        e = 2 * g
        pltpu.async_copy(chunk_src(e + 1), buf1, sem1)
        pltpu.make_async_copy(chunk_src(0), buf0, sem0).wait()
        _reduce_chunk(buf0, acc, row0 + e * CH, starts_v, ends_v)

        @pl.when(e + 2 < NCH)
        def _prefetch():
            pltpu.async_copy(chunk_src(e + 2), buf0, sem0)

        pltpu.make_async_copy(chunk_src(0), buf1, sem1).wait()
        _reduce_chunk(buf1, acc, row0 + (e + 1) * CH, starts_v, ends_v)
        return carry

    lax.fori_loop(0, NCH // 2, pair_body, 0)

    pltpu.sync_copy(acc, spmem.at[sid])
    plsc.subcore_barrier()

    @pl.when(sid < 4)
    def _merge():
        pltpu.sync_copy(spmem.at[0, :, pl.ds(sid * 128, 128)], macc)

        def merge_body(t, carry):
            pltpu.sync_copy(spmem.at[t, :, pl.ds(sid * 128, 128)], mbuf)
            for r in range(NSEG):
                for g in range(8):
                    macc[r, pl.ds(g * 16, 16)] = jnp.maximum(
                        macc[r, pl.ds(g * 16, 16)],
                        mbuf[r, pl.ds(g * 16, 16)],
                    )
            return carry

        lax.fori_loop(1, 16, merge_body, 0)
        pltpu.sync_copy(macc, out_hbm.at[:, pl.ds(col0 + sid * 128, 128)])


BR = 1024


def _tc_block_body(bounds_smem, x_ref, out_ref, *, row_base):
    i = pl.program_id(0)

    @pl.when(i == 0)
    def _init():
        out_ref[...] = jnp.full((NSEG, D), NEG, jnp.float32)

    r0 = row_base + i * BR
    flags = [
        (bounds_smem[0, s] <= r0) & (bounds_smem[1, s] >= r0 + BR)
        for s in range(NSEG)
    ]
    any_full = functools.reduce(jnp.logical_or, flags)
    s_full = functools.reduce(
        jnp.add,
        [jnp.where(flags[s], jnp.int32(s), jnp.int32(0)) for s in range(NSEG)],
    )
    seg_rows = lax.broadcasted_iota(jnp.int32, (NSEG, D), 0)

    @pl.when(any_full)
    def _fast():
        bm = jnp.max(x_ref[...], axis=0)
        out_ref[...] = jnp.where(
            seg_rows == s_full,
            jnp.maximum(out_ref[...], bm[None, :]),
            out_ref[...],
        )

    @pl.when(jnp.logical_not(any_full))
    def _slow():
        rows = r0 + lax.broadcasted_iota(jnp.int32, (BR, 1), 0)
        for s in range(NSEG):
            m = (rows >= bounds_smem[0, s]) & (rows < bounds_smem[1, s])
            contrib = jnp.max(jnp.where(m, x_ref[...], NEG), axis=0)
            out_ref[...] = jnp.where(
                seg_rows == s,
                jnp.maximum(out_ref[...], contrib[None, :]),
                out_ref[...],
            )


def _tc_seg_max(x_full, bounds, row_base):
    blk0 = row_base // BR
    nblk = (x_full.shape[0] - row_base) // BR
    return pl.pallas_call(
        functools.partial(_tc_block_body, row_base=row_base),
        grid=(nblk,),
        in_specs=[
            pl.BlockSpec(memory_space=pltpu.SMEM),
            pl.BlockSpec((BR, D), lambda i: (i + blk0, 0)),
        ],
        out_specs=pl.BlockSpec((NSEG, D), lambda i: (0, 0)),
        out_shape=jax.ShapeDtypeStruct((NSEG, D), jnp.float32),
        compiler_params=pltpu.CompilerParams(
            dimension_semantics=("arbitrary",),
        ),
    )(bounds, x_full)


@jax.jit
def _seg_max(x, starts, ends):
    bounds = jnp.stack([starts, ends])
    p_tc = _tc_seg_max(x, bounds, SC_ROWS)
    mesh = plsc.VectorSubcoreMesh(core_axis_name="c", subcore_axis_name="s")
    p_sc = pl.kernel(
        _body,
        out_type=jax.ShapeDtypeStruct((NSEG, D), jnp.float32),
        mesh=mesh,
        scratch_types=[
            pltpu.VMEM((2, NSEG), jnp.int32),
            pltpu.VMEM((CH, CSC), jnp.float32),
            pltpu.VMEM((CH, CSC), jnp.float32),
            pltpu.VMEM((NSEG, CSC), jnp.float32),
            pltpu.VMEM((NSEG, 128), jnp.float32),
            pltpu.VMEM((NSEG, 128), jnp.float32),
            pltpu.VMEM_SHARED((16, NSEG, CSC), jnp.float32),
            pltpu.SemaphoreType.DMA,
            pltpu.SemaphoreType.DMA,
        ],
    )(x, starts, ends)
    return jnp.maximum(p_sc, p_tc)


def kernel(x, record_len, query, context, Wq, bq, Wc, bc):
    cum = jnp.cumsum(record_len.astype(jnp.int32))
    split = jnp.clip(cum[: NSEG - 1], 0, TOTAL)
    starts = jnp.concatenate([jnp.zeros((1,), jnp.int32), split])
    ends = jnp.concatenate([split, jnp.full((1,), TOTAL, jnp.int32)])
    return _seg_max(x, starts, ends)


# speedup vs baseline: 1.0108x; 1.0108x over previous
import functools

import jax
import jax.numpy as jnp
from jax import lax
from jax.experimental import pallas as pl
from jax.experimental.pallas import tpu as pltpu
from jax.experimental.pallas import tpu_sc as plsc

TOTAL = 16384
NSEG = 16
D = 1024
CSC = 512
SC_ROWS = 2048
RPW = SC_ROWS // 16
CH = 64
NCH = RPW // CH
NG = CSC // 16
NEG = float("-inf")


def _reduce_chunk_ragged(buf, acc, r0, starts_v, ends_v):
    for s in range(NSEG):
        lo = jnp.clip(starts_v[s] - r0, 0, CH)
        hi = jnp.clip(ends_v[s] - r0, 0, CH)

        @pl.when(hi > lo)
        def _process(s=s, lo=lo, hi=hi):
            for h in range(2):
                hg = NG // 2
                c0 = h * hg * 16
                accv = tuple(
                    acc[s, pl.ds(c0 + g * 16, 16)] for g in range(hg)
                )

                def row_body(r, c, c0=c0, hg=hg):
                    return tuple(
                        jnp.maximum(c[g], buf[r, pl.ds(c0 + g * 16, 16)])
                        for g in range(hg)
                    )

                accv = lax.fori_loop(lo, hi, row_body, accv)
                for g in range(hg):
                    acc[s, pl.ds(c0 + g * 16, 16)] = accv[g]


def _reduce_chunk(buf, acc, r0, starts_v, ends_v):
    flags = [
        (starts_v[s] <= r0) & (ends_v[s] >= r0 + CH) for s in range(NSEG)
    ]
    any_full = functools.reduce(jnp.logical_or, flags)
    s_full = functools.reduce(
        jnp.add,
        [jnp.where(flags[s], jnp.int32(s), jnp.int32(0)) for s in range(NSEG)],
    )

    @pl.when(any_full)
    def _fast():
        for h in range(2):
            hg = NG // 2
            c0 = h * hg * 16
            mv = tuple(buf[0, pl.ds(c0 + g * 16, 16)] for g in range(hg))

            @plsc.parallel_loop(1, CH, unroll=4, carry=mv)
            def mv(r, c, c0=c0, hg=hg):
                return tuple(
                    jnp.maximum(c[g], buf[r, pl.ds(c0 + g * 16, 16)])
                    for g in range(hg)
                )

            for g in range(hg):
                a = acc[s_full, pl.ds(c0 + g * 16, 16)]
                acc[s_full, pl.ds(c0 + g * 16, 16)] = jnp.maximum(a, mv[g])

    @pl.when(jnp.logical_not(any_full))
    def _slow():
        _reduce_chunk_ragged(buf, acc, r0, starts_v, ends_v)


def _body(x_hbm, starts_hbm, ends_hbm, out_hbm,
          bnds_s, buf0, buf1, acc, mbuf, macc, spmem, sem0, sem1):
    cid = lax.axis_index("c")
    sid = lax.axis_index("s")
    col0 = cid * CSC
    row0 = sid * RPW

    def chunk_src(i):
        return x_hbm.at[pl.ds(row0 + i * CH, CH), pl.ds(col0, CSC)]

    pltpu.sync_copy(starts_hbm, bnds_s.at[0])
    pltpu.sync_copy(ends_hbm, bnds_s.at[1])
    starts_v = bnds_s[0]
    ends_v = bnds_s[1]

    pltpu.async_copy(chunk_src(0), buf0, sem0)

    neg = jnp.full((16,), NEG, jnp.float32)
    for s in range(NSEG):
        for g in range(NG):
            acc[s, pl.ds(g * 16, 16)] = neg

    def pair_body(g, carry):
        e = 2 * g
        pltpu.async_copy(chunk_src(e + 1), buf1, sem1)
        pltpu.make_async_copy(chunk_src(0), buf0, sem0).wait()
        _reduce_chunk(buf0, acc, row0 + e * CH, starts_v, ends_v)

        @pl.when(e + 2 < NCH)
        def _prefetch():
            pltpu.async_copy(chunk_src(e + 2), buf0, sem0)

        pltpu.make_async_copy(chunk_src(0), buf1, sem1).wait()
        _reduce_chunk(buf1, acc, row0 + (e + 1) * CH, starts_v, ends_v)
        return carry

    lax.fori_loop(0, NCH // 2, pair_body, 0)

    pltpu.sync_copy(acc, spmem.at[sid])
    plsc.subcore_barrier()

    @pl.when(sid < 4)
    def _merge():
        pltpu.sync_copy(spmem.at[0, :, pl.ds(sid * 128, 128)], macc)

        def merge_body(t, carry):
            pltpu.sync_copy(spmem.at[t, :, pl.ds(sid * 128, 128)], mbuf)
            for r in range(NSEG):
                for g in range(8):
                    macc[r, pl.ds(g * 16, 16)] = jnp.maximum(
                        macc[r, pl.ds(g * 16, 16)],
                        mbuf[r, pl.ds(g * 16, 16)],
                    )
            return carry

        lax.fori_loop(1, 16, merge_body, 0)
        pltpu.sync_copy(macc, out_hbm.at[:, pl.ds(col0 + sid * 128, 128)])


BR = 512


def _tc_block_body(bounds_smem, x_ref, out_ref, *, row_base):
    i = pl.program_id(0)

    @pl.when(i == 0)
    def _init():
        out_ref[...] = jnp.full((NSEG, D), NEG, jnp.float32)

    r0 = row_base + i * BR
    flags = [
        (bounds_smem[0, s] <= r0) & (bounds_smem[1, s] >= r0 + BR)
        for s in range(NSEG)
    ]
    any_full = functools.reduce(jnp.logical_or, flags)
    s_full = functools.reduce(
        jnp.add,
        [jnp.where(flags[s], jnp.int32(s), jnp.int32(0)) for s in range(NSEG)],
    )
    seg_rows = lax.broadcasted_iota(jnp.int32, (NSEG, D), 0)

    @pl.when(any_full)
    def _fast():
        bm = jnp.max(x_ref[...], axis=0)
        out_ref[...] = jnp.where(
            seg_rows == s_full,
            jnp.maximum(out_ref[...], bm[None, :]),
            out_ref[...],
        )

    @pl.when(jnp.logical_not(any_full))
    def _slow():
        rows = r0 + lax.broadcasted_iota(jnp.int32, (BR, 1), 0)
        for s in range(NSEG):
            m = (rows >= bounds_smem[0, s]) & (rows < bounds_smem[1, s])
            contrib = jnp.max(jnp.where(m, x_ref[...], NEG), axis=0)
            out_ref[...] = jnp.where(
                seg_rows == s,
                jnp.maximum(out_ref[...], contrib[None, :]),
                out_ref[...],
            )


def _tc_seg_max(x_full, bounds, row_base):
    blk0 = row_base // BR
    nblk = (x_full.shape[0] - row_base) // BR
    return pl.pallas_call(
        functools.partial(_tc_block_body, row_base=row_base),
        grid=(nblk,),
        in_specs=[
            pl.BlockSpec(memory_space=pltpu.SMEM),
            pl.BlockSpec((BR, D), lambda i: (i + blk0, 0)),
        ],
        out_specs=pl.BlockSpec((NSEG, D), lambda i: (0, 0)),
        out_shape=jax.ShapeDtypeStruct((NSEG, D), jnp.float32),
        compiler_params=pltpu.CompilerParams(
            dimension_semantics=("arbitrary",),
        ),
    )(bounds, x_full)


@jax.jit
def _seg_max(x, starts, ends):
    bounds = jnp.stack([starts, ends])
    p_tc = _tc_seg_max(x, bounds, SC_ROWS)
    mesh = plsc.VectorSubcoreMesh(core_axis_name="c", subcore_axis_name="s")
    p_sc = pl.kernel(
        _body,
        out_type=jax.ShapeDtypeStruct((NSEG, D), jnp.float32),
        mesh=mesh,
        scratch_types=[
            pltpu.VMEM((2, NSEG), jnp.int32),
            pltpu.VMEM((CH, CSC), jnp.float32),
            pltpu.VMEM((CH, CSC), jnp.float32),
            pltpu.VMEM((NSEG, CSC), jnp.float32),
            pltpu.VMEM((NSEG, 128), jnp.float32),
            pltpu.VMEM((NSEG, 128), jnp.float32),
            pltpu.VMEM_SHARED((16, NSEG, CSC), jnp.float32),
            pltpu.SemaphoreType.DMA,
            pltpu.SemaphoreType.DMA,
        ],
    )(x, starts, ends)
    return jnp.maximum(p_sc, p_tc)


def kernel(x, record_len, query, context, Wq, bq, Wc, bc):
    cum = jnp.cumsum(record_len.astype(jnp.int32))
    split = jnp.clip(cum[: NSEG - 1], 0, TOTAL)
    starts = jnp.concatenate([jnp.zeros((1,), jnp.int32), split])
    ends = jnp.concatenate([split, jnp.full((1,), TOTAL, jnp.int32)])
    return _seg_max(x, starts, ends)
